# baseline (speedup 1.0000x reference)
import functools

import jax
import jax.numpy as jnp
import numpy as np
from jax import lax
from jax.experimental import pallas as pl
from jax.experimental.pallas import tpu as pltpu
from jax.experimental.pallas import tpu_sc as plsc

BN_EPS = 1e-5
NC = 2
NS = 16
CH = 64


def _sc_mesh():
    return plsc.VectorSubcoreMesh(
        core_axis_name="c", subcore_axis_name="s", num_cores=NC, num_subcores=NS
    )


def _make_deg_kernel(N_pad, W, K):
    ZR = N_pad // NS

    @functools.partial(
        pl.kernel,
        out_type=jax.ShapeDtypeStruct((NC, N_pad, W), jnp.float32),
        mesh=_sc_mesh(),
        scratch_types=[
            pltpu.VMEM((K, CH), jnp.int32),
            pltpu.VMEM((CH, W), jnp.float32),
            pltpu.VMEM_SHARED((N_pad, W), jnp.float32),
        ],
    )
    def deg_kernel(dst_hbm, ones_hbm, zeros_hbm, out_hbm, dst_v, ones_v, acc):
        c = lax.axis_index("c")
        s = lax.axis_index("s")
        wid = c * NS + s
        pltpu.sync_copy(zeros_hbm, acc.at[pl.ds(s * ZR, ZR)])
        pltpu.sync_copy(ones_hbm, ones_v)
        pltpu.sync_copy(dst_hbm.at[pl.ds(wid * K, K)], dst_v)
        plsc.subcore_barrier()

        def body(j, carry):
            pltpu.sync_copy(ones_v, acc.at[dst_v.at[j]], add=True)
            return carry

        lax.fori_loop(0, K, body, 0)
        plsc.subcore_barrier()
        pltpu.sync_copy(
            acc.at[pl.ds(s * ZR, ZR)], out_hbm.at[c].at[pl.ds(s * ZR, ZR)]
        )

    return deg_kernel


def _make_scatter_kernel(N_pad, D, K):
    ZR = N_pad // NS

    NB = 4
    PH = 4
    assert K % (PH * NB) == 0
    K2 = K // PH

    @functools.partial(
        pl.kernel,
        out_type=jax.ShapeDtypeStruct((NC, N_pad, D), jnp.float32),
        mesh=_sc_mesh(),
        scratch_types=[
            pltpu.VMEM((K2, CH), jnp.int32),
            pltpu.VMEM((K2, CH), jnp.int32),
            [pltpu.VMEM((CH, D), jnp.float32) for _ in range(NB)],
            [pltpu.SemaphoreType.DMA for _ in range(NB)],
            pltpu.VMEM_SHARED((N_pad, D), jnp.float32),
        ],
    )
    def scatter_kernel(h_hbm, src_hbm, dst_hbm, zeros_hbm, out_hbm,
                       src_v, dst_v, rows_v, sems, acc):
        c = lax.axis_index("c")
        s = lax.axis_index("s")
        wid = c * NS + s
        pltpu.sync_copy(zeros_hbm, acc.at[pl.ds(s * ZR, ZR)])
        plsc.subcore_barrier()

        def gather(j, b):
            return pltpu.make_async_copy(
                h_hbm.at[src_v.at[j]], rows_v[b], sems[b]
            )

        for phase in range(PH):
            base = wid * K + phase * K2
            pltpu.sync_copy(src_hbm.at[pl.ds(base, K2)], src_v)
            pltpu.sync_copy(dst_hbm.at[pl.ds(base, K2)], dst_v)

            for b in range(NB):
                gather(b, b).start()

            def body(jj, carry):
                for b in range(NB):
                    j = jj * NB + b
                    gather(j, b).wait()
                    pltpu.sync_copy(rows_v[b], acc.at[dst_v.at[j]], add=True)
                    gather(j + NB, b).start()
                return carry

            lax.fori_loop(0, K2 // NB - 1, body, 0)
            for b in range(NB):
                j = K2 - NB + b
                gather(j, b).wait()
                pltpu.sync_copy(rows_v[b], acc.at[dst_v.at[j]], add=True)

        plsc.subcore_barrier()
        pltpu.sync_copy(
            acc.at[pl.ds(s * ZR, ZR)], out_hbm.at[c].at[pl.ds(s * ZR, ZR)]
        )

    return scatter_kernel


def _tc_prologue(x, W1, deg_parts):
    N, D = x.shape
    H = W1.shape[1]

    def body(x_ref, w_ref, dp_ref, hp_ref, dis_ref):
        deg = dp_ref[0, 0:N, 0:1] + dp_ref[1, 0:N, 0:1] + 1.0
        dis = lax.rsqrt(deg)
        dis_ref[...] = dis
        hp_ref[...] = (
            jnp.dot(x_ref[...], w_ref[...], preferred_element_type=jnp.float32)
            * dis
        )

    return pl.pallas_call(
        body,
        out_shape=(
            jax.ShapeDtypeStruct((N, H), jnp.float32),
            jax.ShapeDtypeStruct((N, 1), jnp.float32),
        ),
    )(x, W1, deg_parts)


def _tc_mid(S, hp, dis, b, g, be, Wn):
    N, H = hp.shape
    scale = float(1.0 / np.sqrt(1.0 + BN_EPS))

    def body(s_ref, hp_ref, dis_ref, b_ref, g_ref, be_ref, w_ref, out_ref):
        dis = dis_ref[...]
        S01 = s_ref[0, 0:N] + s_ref[1, 0:N]
        conv = (S01 + hp_ref[...]) * dis + b_ref[...]
        z = jnp.maximum(conv * (g_ref[...] * scale) + be_ref[...], 0.0)
        out_ref[...] = (
            jnp.dot(z, w_ref[...], preferred_element_type=jnp.float32) * dis
        )

    return pl.pallas_call(
        body, out_shape=jax.ShapeDtypeStruct((N, H), jnp.float32)
    )(S, hp, dis, b, g, be, Wn)


def _tc_final(S, hp, dis, b, g, be):
    N, H = hp.shape
    scale = float(1.0 / np.sqrt(1.0 + BN_EPS))

    def body(s_ref, hp_ref, dis_ref, b_ref, g_ref, be_ref, h_ref, m_ref):
        dis = dis_ref[...]
        S01 = s_ref[0, 0:N] + s_ref[1, 0:N]
        conv = (S01 + hp_ref[...]) * dis + b_ref[...]
        z = jnp.maximum(conv * (g_ref[...] * scale) + be_ref[...], 0.0)
        h_ref[...] = z
        m_ref[...] = jnp.mean(z, axis=0, keepdims=True)

    return pl.pallas_call(
        body,
        out_shape=(
            jax.ShapeDtypeStruct((N, H), jnp.float32),
            jax.ShapeDtypeStruct((1, H), jnp.float32),
        ),
    )(S, hp, dis, b, g, be)


def kernel(x, edge_index, W1, b1, g1, be1, W2, b2, g2, be2, W3, b3, g3, be3):
    N, D = x.shape
    H = W1.shape[1]
    E = edge_index.shape[1]
    NW = NC * NS

    chunks = -(-E // CH)
    K = -(-chunks // NW)
    K = -(-K // 16) * 16
    E_pad = NW * K * CH
    N_pad = -(-N // (8 * NS)) * (8 * NS)
    if N_pad == N:
        N_pad += 8 * NS

    src, dst = lax.sort((edge_index[0], edge_index[1]), num_keys=1)
    pad = E_pad - E
    if pad:
        src = jnp.concatenate([src, jnp.zeros((pad,), jnp.int32)])
        dst = jnp.concatenate(
            [dst, N + (jnp.arange(pad, dtype=jnp.int32) % (N_pad - N))]
        )
    src2 = src.reshape(NW * K, CH)
    dst2 = dst.reshape(NW * K, CH)

    ZR = N_pad // NS
    ones_rows = jnp.ones((CH, D), jnp.float32)
    zrows = jnp.zeros((ZR, D), jnp.float32)

    deg_parts = _make_deg_kernel(N_pad, D, K)(dst2, ones_rows, zrows)
    scat = _make_scatter_kernel(N_pad, D, K)

    b1r, g1r, be1r = b1.reshape(1, H), g1.reshape(1, H), be1.reshape(1, H)
    b2r, g2r, be2r = b2.reshape(1, H), g2.reshape(1, H), be2.reshape(1, H)
    b3r, g3r, be3r = b3.reshape(1, H), g3.reshape(1, H), be3.reshape(1, H)

    hp1, dis = _tc_prologue(x, W1, deg_parts)
    S1 = scat(hp1, src2, dst2, zrows)
    hp2 = _tc_mid(S1, hp1, dis, b1r, g1r, be1r, W2)
    S2 = scat(hp2, src2, dst2, zrows)
    hp3 = _tc_mid(S2, hp2, dis, b2r, g2r, be2r, W3)
    S3 = scat(hp3, src2, dst2, zrows)
    h, gemb = _tc_final(S3, hp3, dis, b3r, g3r, be3r)
    return (h, gemb)


# speedup vs baseline: 1.4236x; 1.4236x over previous
import functools

import jax
import jax.numpy as jnp
import numpy as np
from jax import lax
from jax.experimental import pallas as pl
from jax.experimental.pallas import tpu as pltpu
from jax.experimental.pallas import tpu_sc as plsc

BN_EPS = 1e-5
NC = 2
NS = 16
CH = 64
SPLIT0 = 0.5


def _sc_mesh():
    return plsc.VectorSubcoreMesh(
        core_axis_name="c", subcore_axis_name="s", num_cores=NC, num_subcores=NS
    )


def _make_deg_kernel(N_pad, W, K):
    ZR = N_pad // NS

    @functools.partial(
        pl.kernel,
        out_type=jax.ShapeDtypeStruct((NC, N_pad, W), jnp.float32),
        mesh=_sc_mesh(),
        scratch_types=[
            pltpu.VMEM((K, CH), jnp.int32),
            pltpu.VMEM((CH, W), jnp.float32),
            pltpu.VMEM_SHARED((N_pad, W), jnp.float32),
        ],
    )
    def deg_kernel(dst_hbm, ones_hbm, zeros_hbm, out_hbm, dst_v, ones_v, acc):
        c = lax.axis_index("c")
        s = lax.axis_index("s")
        wid = c * NS + s
        pltpu.sync_copy(zeros_hbm, acc.at[pl.ds(s * ZR, ZR)])
        pltpu.sync_copy(ones_hbm, ones_v)
        pltpu.sync_copy(dst_hbm.at[pl.ds(wid * K, K)], dst_v)
        plsc.subcore_barrier()

        def body(j, carry):
            pltpu.sync_copy(ones_v, acc.at[dst_v.at[j]], add=True)
            return carry

        lax.fori_loop(0, K, body, 0)
        plsc.subcore_barrier()
        pltpu.sync_copy(
            acc.at[pl.ds(s * ZR, ZR)], out_hbm.at[c].at[pl.ds(s * ZR, ZR)]
        )

    return deg_kernel


def _make_scatter_kernel(N_pad, D, K0, K1):
    ZR = N_pad // NS

    NB = 4
    PH = 4
    assert K0 % (PH * NB) == 0 and K1 % (PH * NB) == 0
    assert (K0 // PH) % 8 == 0 and (K1 // PH) % 8 == 0
    K2M = max(K0, K1) // PH

    @functools.partial(
        pl.kernel,
        out_type=jax.ShapeDtypeStruct((NC, N_pad, D), jnp.float32),
        mesh=_sc_mesh(),
        scratch_types=[
            pltpu.VMEM((K2M, CH), jnp.int32),
            pltpu.VMEM((K2M, CH), jnp.int32),
            [pltpu.VMEM((CH, D), jnp.float32) for _ in range(NB)],
            [pltpu.SemaphoreType.DMA for _ in range(NB)],
            pltpu.VMEM_SHARED((N_pad, D), jnp.float32),
        ],
    )
    def scatter_kernel(h_hbm, src_hbm, dst_hbm, zeros_hbm, out_hbm,
                       src_v, dst_v, rows_v, sems, acc):
        c = lax.axis_index("c")
        s = lax.axis_index("s")
        pltpu.sync_copy(zeros_hbm, acc.at[pl.ds(s * ZR, ZR)])
        plsc.subcore_barrier()

        def gather(j, b):
            return pltpu.make_async_copy(
                h_hbm.at[src_v.at[j]], rows_v[b], sems[b]
            )

        def run_core(Kc, base0):
            K2 = Kc // PH
            for phase in range(PH):
                base = base0 + phase * K2
                pltpu.sync_copy(
                    src_hbm.at[pl.ds(base, K2)], src_v.at[pl.ds(0, K2)]
                )
                pltpu.sync_copy(
                    dst_hbm.at[pl.ds(base, K2)], dst_v.at[pl.ds(0, K2)]
                )

                for b in range(NB):
                    gather(b, b).start()

                def body(jj, carry):
                    for b in range(NB):
                        j = jj * NB + b
                        gather(j, b).wait()
                        pltpu.sync_copy(
                            rows_v[b], acc.at[dst_v.at[j]], add=True
                        )
                        gather(j + NB, b).start()
                    return carry

                lax.fori_loop(0, K2 // NB - 1, body, 0)
                for b in range(NB):
                    j = K2 - NB + b
                    gather(j, b).wait()
                    pltpu.sync_copy(rows_v[b], acc.at[dst_v.at[j]], add=True)

        @pl.when(c == 0)
        def _():
            run_core(K0, s * K0)

        @pl.when(c == 1)
        def _():
            run_core(K1, NS * K0 + s * K1)

        plsc.subcore_barrier()
        pltpu.sync_copy(
            acc.at[pl.ds(s * ZR, ZR)], out_hbm.at[c].at[pl.ds(s * ZR, ZR)]
        )

    return scatter_kernel


def _tc_prologue(x, W1, deg_parts):
    N, D = x.shape
    H = W1.shape[1]

    def body(x_ref, w_ref, dp_ref, hp_ref, dis_ref):
        deg = dp_ref[0, 0:N, 0:1] + dp_ref[1, 0:N, 0:1] + 1.0
        dis = lax.rsqrt(deg)
        dis_ref[...] = dis
        hp_ref[...] = (
            jnp.dot(x_ref[...], w_ref[...], preferred_element_type=jnp.float32)
            * dis
        )

    return pl.pallas_call(
        body,
        out_shape=(
            jax.ShapeDtypeStruct((N, H), jnp.float32),
            jax.ShapeDtypeStruct((N, 1), jnp.float32),
        ),
    )(x, W1, deg_parts)


def _tc_mid(S, hp, dis, b, g, be, Wn):
    N, H = hp.shape
    scale = float(1.0 / np.sqrt(1.0 + BN_EPS))

    def body(s_ref, hp_ref, dis_ref, b_ref, g_ref, be_ref, w_ref, out_ref):
        dis = dis_ref[...]
        S01 = s_ref[0, 0:N] + s_ref[1, 0:N]
        conv = (S01 + hp_ref[...]) * dis + b_ref[...]
        z = jnp.maximum(conv * (g_ref[...] * scale) + be_ref[...], 0.0)
        out_ref[...] = (
            jnp.dot(z, w_ref[...], preferred_element_type=jnp.float32) * dis
        )

    return pl.pallas_call(
        body, out_shape=jax.ShapeDtypeStruct((N, H), jnp.float32)
    )(S, hp, dis, b, g, be, Wn)


def _tc_final(S, hp, dis, b, g, be):
    N, H = hp.shape
    scale = float(1.0 / np.sqrt(1.0 + BN_EPS))

    def body(s_ref, hp_ref, dis_ref, b_ref, g_ref, be_ref, h_ref, m_ref):
        dis = dis_ref[...]
        S01 = s_ref[0, 0:N] + s_ref[1, 0:N]
        conv = (S01 + hp_ref[...]) * dis + b_ref[...]
        z = jnp.maximum(conv * (g_ref[...] * scale) + be_ref[...], 0.0)
        h_ref[...] = z
        m_ref[...] = jnp.mean(z, axis=0, keepdims=True)

    return pl.pallas_call(
        body,
        out_shape=(
            jax.ShapeDtypeStruct((N, H), jnp.float32),
            jax.ShapeDtypeStruct((1, H), jnp.float32),
        ),
    )(S, hp, dis, b, g, be)


def kernel(x, edge_index, W1, b1, g1, be1, W2, b2, g2, be2, W3, b3, g3, be3):
    N, D = x.shape
    H = W1.shape[1]
    E = edge_index.shape[1]
    NW = NC * NS

    chunks = -(-E // CH)
    K = -(-chunks // NW)
    K = -(-K // 16) * 16
    E_pad = NW * K * CH
    N_pad = -(-N // (8 * NS)) * (8 * NS)
    if N_pad == N:
        N_pad += 8 * NS

    src = edge_index[0]
    dst = edge_index[1]
    pad = E_pad - E
    if pad:
        src = jnp.concatenate([src, jnp.zeros((pad,), jnp.int32)])
        dst = jnp.concatenate(
            [dst, N + (jnp.arange(pad, dtype=jnp.int32) % (N_pad - N))]
        )
    src2 = src.reshape(NW * K, CH)
    dst2 = dst.reshape(NW * K, CH)

    ZR = N_pad // NS
    ones_rows = jnp.ones((CH, D), jnp.float32)
    zrows = jnp.zeros((ZR, D), jnp.float32)

    deg_parts = _make_deg_kernel(N_pad, D, K)(dst2, ones_rows, zrows)
    K0 = (-(-int(2 * K * SPLIT0) // 64) * 64) if SPLIT0 != 0.5 else K
    K1 = 2 * K - K0
    scat = _make_scatter_kernel(N_pad, D, K0, K1)

    b1r, g1r, be1r = b1.reshape(1, H), g1.reshape(1, H), be1.reshape(1, H)
    b2r, g2r, be2r = b2.reshape(1, H), g2.reshape(1, H), be2.reshape(1, H)
    b3r, g3r, be3r = b3.reshape(1, H), g3.reshape(1, H), be3.reshape(1, H)

    hp1, dis = _tc_prologue(x, W1, deg_parts)
    S1 = scat(hp1, src2, dst2, zrows)
    hp2 = _tc_mid(S1, hp1, dis, b1r, g1r, be1r, W2)
    S2 = scat(hp2, src2, dst2, zrows)
    hp3 = _tc_mid(S2, hp2, dis, b2r, g2r, be2r, W3)
    S3 = scat(hp3, src2, dst2, zrows)
    h, gemb = _tc_final(S3, hp3, dis, b3r, g3r, be3r)
    return (h, gemb)


# speedup vs baseline: 1.4418x; 1.0127x over previous
import functools

import jax
import jax.numpy as jnp
import numpy as np
from jax import lax
from jax.experimental import pallas as pl
from jax.experimental.pallas import tpu as pltpu
from jax.experimental.pallas import tpu_sc as plsc

BN_EPS = 1e-5
NC = 2
NS = 16
CH = 64
SPLIT0 = 0.6


def _sc_mesh():
    return plsc.VectorSubcoreMesh(
        core_axis_name="c", subcore_axis_name="s", num_cores=NC, num_subcores=NS
    )


def _make_deg_kernel(N_pad, W, K):
    ZR = N_pad // NS

    @functools.partial(
        pl.kernel,
        out_type=jax.ShapeDtypeStruct((NC, N_pad, W), jnp.float32),
        mesh=_sc_mesh(),
        scratch_types=[
            pltpu.VMEM((K, CH), jnp.int32),
            pltpu.VMEM((CH, W), jnp.float32),
            pltpu.VMEM_SHARED((N_pad, W), jnp.float32),
        ],
    )
    def deg_kernel(dst_hbm, ones_hbm, zeros_hbm, out_hbm, dst_v, ones_v, acc):
        c = lax.axis_index("c")
        s = lax.axis_index("s")
        wid = c * NS + s
        pltpu.sync_copy(zeros_hbm, acc.at[pl.ds(s * ZR, ZR)])
        pltpu.sync_copy(ones_hbm, ones_v)
        pltpu.sync_copy(dst_hbm.at[pl.ds(wid * K, K)], dst_v)
        plsc.subcore_barrier()

        def body(j, carry):
            pltpu.sync_copy(ones_v, acc.at[dst_v.at[j]], add=True)
            return carry

        lax.fori_loop(0, K, body, 0)
        plsc.subcore_barrier()
        pltpu.sync_copy(
            acc.at[pl.ds(s * ZR, ZR)], out_hbm.at[c].at[pl.ds(s * ZR, ZR)]
        )

    return deg_kernel


def _make_scatter_kernel(N_pad, D, K0, K1):
    ZR = N_pad // NS

    NB = 4
    PH = 4
    assert K0 % (PH * NB) == 0 and K1 % (PH * NB) == 0
    assert (K0 // PH) % 8 == 0 and (K1 // PH) % 8 == 0
    K2M = max(K0, K1) // PH

    @functools.partial(
        pl.kernel,
        out_type=jax.ShapeDtypeStruct((NC, N_pad, D), jnp.float32),
        mesh=_sc_mesh(),
        scratch_types=[
            pltpu.VMEM((K2M, CH), jnp.int32),
            pltpu.VMEM((K2M, CH), jnp.int32),
            [pltpu.VMEM((CH, D), jnp.float32) for _ in range(NB)],
            [pltpu.SemaphoreType.DMA for _ in range(NB)],
            pltpu.VMEM_SHARED((N_pad, D), jnp.float32),
        ],
    )
    def scatter_kernel(h_hbm, src_hbm, dst_hbm, zeros_hbm, out_hbm,
                       src_v, dst_v, rows_v, sems, acc):
        c = lax.axis_index("c")
        s = lax.axis_index("s")
        pltpu.sync_copy(zeros_hbm, acc.at[pl.ds(s * ZR, ZR)])
        plsc.subcore_barrier()

        def gather(j, b):
            return pltpu.make_async_copy(
                h_hbm.at[src_v.at[j]], rows_v[b], sems[b]
            )

        def run_core(Kc, base0):
            K2 = Kc // PH
            for phase in range(PH):
                base = base0 + phase * K2
                pltpu.sync_copy(
                    src_hbm.at[pl.ds(base, K2)], src_v.at[pl.ds(0, K2)]
                )
                pltpu.sync_copy(
                    dst_hbm.at[pl.ds(base, K2)], dst_v.at[pl.ds(0, K2)]
                )

                for b in range(NB):
                    gather(b, b).start()

                def body(jj, carry):
                    for b in range(NB):
                        j = jj * NB + b
                        gather(j, b).wait()
                        pltpu.sync_copy(
                            rows_v[b], acc.at[dst_v.at[j]], add=True
                        )
                        gather(j + NB, b).start()
                    return carry

                lax.fori_loop(0, K2 // NB - 1, body, 0)
                for b in range(NB):
                    j = K2 - NB + b
                    gather(j, b).wait()
                    pltpu.sync_copy(rows_v[b], acc.at[dst_v.at[j]], add=True)

        @pl.when(c == 0)
        def _():
            run_core(K0, s * K0)

        @pl.when(c == 1)
        def _():
            run_core(K1, NS * K0 + s * K1)

        plsc.subcore_barrier()
        pltpu.sync_copy(
            acc.at[pl.ds(s * ZR, ZR)], out_hbm.at[c].at[pl.ds(s * ZR, ZR)]
        )

    return scatter_kernel


def _tc_prologue(x, W1, deg_parts):
    N, D = x.shape
    H = W1.shape[1]

    def body(x_ref, w_ref, dp_ref, hp_ref, dis_ref):
        deg = dp_ref[0, 0:N, 0:1] + dp_ref[1, 0:N, 0:1] + 1.0
        dis = lax.rsqrt(deg)
        dis_ref[...] = dis
        hp_ref[...] = (
            jnp.dot(x_ref[...], w_ref[...], preferred_element_type=jnp.float32)
            * dis
        )

    return pl.pallas_call(
        body,
        out_shape=(
            jax.ShapeDtypeStruct((N, H), jnp.float32),
            jax.ShapeDtypeStruct((N, 1), jnp.float32),
        ),
    )(x, W1, deg_parts)


def _tc_mid(S, hp, dis, b, g, be, Wn):
    N, H = hp.shape
    scale = float(1.0 / np.sqrt(1.0 + BN_EPS))

    def body(s_ref, hp_ref, dis_ref, b_ref, g_ref, be_ref, w_ref, out_ref):
        dis = dis_ref[...]
        S01 = s_ref[0, 0:N] + s_ref[1, 0:N]
        conv = (S01 + hp_ref[...]) * dis + b_ref[...]
        z = jnp.maximum(conv * (g_ref[...] * scale) + be_ref[...], 0.0)
        out_ref[...] = (
            jnp.dot(z, w_ref[...], preferred_element_type=jnp.float32) * dis
        )

    return pl.pallas_call(
        body, out_shape=jax.ShapeDtypeStruct((N, H), jnp.float32)
    )(S, hp, dis, b, g, be, Wn)


def _tc_final(S, hp, dis, b, g, be):
    N, H = hp.shape
    scale = float(1.0 / np.sqrt(1.0 + BN_EPS))

    def body(s_ref, hp_ref, dis_ref, b_ref, g_ref, be_ref, h_ref, m_ref):
        dis = dis_ref[...]
        S01 = s_ref[0, 0:N] + s_ref[1, 0:N]
        conv = (S01 + hp_ref[...]) * dis + b_ref[...]
        z = jnp.maximum(conv * (g_ref[...] * scale) + be_ref[...], 0.0)
        h_ref[...] = z
        m_ref[...] = jnp.mean(z, axis=0, keepdims=True)

    return pl.pallas_call(
        body,
        out_shape=(
            jax.ShapeDtypeStruct((N, H), jnp.float32),
            jax.ShapeDtypeStruct((1, H), jnp.float32),
        ),
    )(S, hp, dis, b, g, be)


def kernel(x, edge_index, W1, b1, g1, be1, W2, b2, g2, be2, W3, b3, g3, be3):
    N, D = x.shape
    H = W1.shape[1]
    E = edge_index.shape[1]
    NW = NC * NS

    chunks = -(-E // CH)
    K = -(-chunks // NW)
    K = -(-K // 16) * 16
    E_pad = NW * K * CH
    N_pad = -(-N // (8 * NS)) * (8 * NS)
    if N_pad == N:
        N_pad += 8 * NS

    src = edge_index[0]
    dst = edge_index[1]
    pad = E_pad - E
    if pad:
        src = jnp.concatenate([src, jnp.zeros((pad,), jnp.int32)])
        dst = jnp.concatenate(
            [dst, N + (jnp.arange(pad, dtype=jnp.int32) % (N_pad - N))]
        )
    src2 = src.reshape(NW * K, CH)
    dst2 = dst.reshape(NW * K, CH)

    ZR = N_pad // NS
    ones_rows = jnp.ones((CH, D), jnp.float32)
    zrows = jnp.zeros((ZR, D), jnp.float32)

    deg_parts = _make_deg_kernel(N_pad, D, K)(dst2, ones_rows, zrows)
    K0 = (-(-int(2 * K * SPLIT0) // 64) * 64) if SPLIT0 != 0.5 else K
    K1 = 2 * K - K0
    scat = _make_scatter_kernel(N_pad, D, K0, K1)

    b1r, g1r, be1r = b1.reshape(1, H), g1.reshape(1, H), be1.reshape(1, H)
    b2r, g2r, be2r = b2.reshape(1, H), g2.reshape(1, H), be2.reshape(1, H)
    b3r, g3r, be3r = b3.reshape(1, H), g3.reshape(1, H), be3.reshape(1, H)

    hp1, dis = _tc_prologue(x, W1, deg_parts)
    S1 = scat(hp1, src2, dst2, zrows)
    hp2 = _tc_mid(S1, hp1, dis, b1r, g1r, be1r, W2)
    S2 = scat(hp2, src2, dst2, zrows)
    hp3 = _tc_mid(S2, hp2, dis, b2r, g2r, be2r, W3)
    S3 = scat(hp3, src2, dst2, zrows)
    h, gemb = _tc_final(S3, hp3, dis, b3r, g3r, be3r)
    return (h, gemb)
